# Initial kernel scaffold; baseline (speedup 1.0000x reference)
import jax
import jax.numpy as jnp
from jax.experimental import pallas as pl


def kernel(words, pretrained_weight, trainable_weight):
    raise NotImplementedError("write your pallas kernel here")


# speedup vs baseline: 3.0449x; 3.0449x over previous
import functools

import jax
import jax.numpy as jnp
from jax import lax
from jax.experimental import pallas as pl
from jax.experimental.pallas import tpu as pltpu
from jax.experimental.pallas import tpu_sc as plsc

VOCAB = 100000
DIM = 128
B = 4096
L = 50

NC = 2
NS = 16
NW = NC * NS

N_IDX = B * L
IDX_PER_W = N_IDX // NW
CHUNK = 128
CHUNKS_PER_W = IDX_PER_W // CHUNK


def _body(words_hbm, pre_hbm, trn_hbm, out_hbm, idx_v, buf_a, buf_b, sem):
    wid = lax.axis_index("s") * NC + lax.axis_index("c")
    base = wid * IDX_PER_W

    pltpu.sync_copy(words_hbm.at[wid], idx_v)

    def chunk_body(j, carry):
        idx = idx_v.at[j]
        cp_a = pltpu.async_copy(pre_hbm.at[idx], buf_a, sem)
        cp_b = pltpu.async_copy(trn_hbm.at[idx], buf_b, sem)
        cp_a.wait()
        cp_b.wait()

        def add_row(r, c2):
            for c in range(DIM // 16):
                sl = pl.ds(c * 16, 16)
                buf_a[r, sl] = buf_a[r, sl] + buf_b[r, sl]
            return c2

        lax.fori_loop(0, CHUNK, add_row, 0, unroll=2)
        pltpu.sync_copy(buf_a, out_hbm.at[pl.ds(base + j * CHUNK, CHUNK)])
        return carry

    lax.fori_loop(0, CHUNKS_PER_W, chunk_body, 0)


@jax.jit
def _run(words2d, pre, trn):
    mesh = plsc.VectorSubcoreMesh(
        core_axis_name="c", subcore_axis_name="s", num_cores=NC, num_subcores=NS
    )
    f = pl.kernel(
        _body,
        out_type=jax.ShapeDtypeStruct((N_IDX, DIM), jnp.float32),
        mesh=mesh,
        scratch_types=[
            pltpu.VMEM((CHUNKS_PER_W, CHUNK), jnp.int32),
            pltpu.VMEM((CHUNK, DIM), jnp.float32),
            pltpu.VMEM((CHUNK, DIM), jnp.float32),
            pltpu.SemaphoreType.DMA,
        ],
    )
    return f(words2d, pre, trn)


def kernel(words, pretrained_weight, trainable_weight):
    words2d = words.reshape(NW, CHUNKS_PER_W, CHUNK)
    out = _run(words2d, pretrained_weight, trainable_weight)
    return out.reshape(B, L, DIM)


# speedup vs baseline: 4.5003x; 1.4780x over previous
import functools

import jax
import jax.numpy as jnp
from jax import lax
from jax.experimental import pallas as pl
from jax.experimental.pallas import tpu as pltpu
from jax.experimental.pallas import tpu_sc as plsc

VOCAB = 100000
DIM = 128
B = 4096
L = 50

NC = 2
NS = 16
NW = NC * NS

N_IDX = B * L
IDX_PER_W = N_IDX // NW
CHUNK = 128
CHUNKS_PER_W = IDX_PER_W // CHUNK


def _body(words_hbm, pre_hbm, trn_hbm, out_hbm, idx_v, buf_a, buf_b, sem):
    wid = lax.axis_index("s") * NC + lax.axis_index("c")
    base = wid * IDX_PER_W

    pltpu.sync_copy(words_hbm.at[wid], idx_v)

    def chunk_body(j, carry):
        idx = idx_v.at[j]
        cp_a = pltpu.async_copy(pre_hbm.at[idx], buf_a, sem)
        cp_a.wait()
        cp_b = pltpu.async_copy(trn_hbm.at[idx], buf_a, sem, add=True)
        cp_b.wait()
        pltpu.sync_copy(buf_a, out_hbm.at[pl.ds(base + j * CHUNK, CHUNK)])
        return carry

    lax.fori_loop(0, CHUNKS_PER_W, chunk_body, 0)


@jax.jit
def _run(words2d, pre, trn):
    mesh = plsc.VectorSubcoreMesh(
        core_axis_name="c", subcore_axis_name="s", num_cores=NC, num_subcores=NS
    )
    f = pl.kernel(
        _body,
        out_type=jax.ShapeDtypeStruct((N_IDX, DIM), jnp.float32),
        mesh=mesh,
        scratch_types=[
            pltpu.VMEM((CHUNKS_PER_W, CHUNK), jnp.int32),
            pltpu.VMEM((CHUNK, DIM), jnp.float32),
            pltpu.VMEM((CHUNK, DIM), jnp.float32),
            pltpu.SemaphoreType.DMA,
        ],
    )
    return f(words2d, pre, trn)


def kernel(words, pretrained_weight, trainable_weight):
    words2d = words.reshape(NW, CHUNKS_PER_W, CHUNK)
    out = _run(words2d, pretrained_weight, trainable_weight)
    return out.reshape(B, L, DIM)


# speedup vs baseline: 5.1379x; 1.1417x over previous
import functools

import jax
import jax.numpy as jnp
from jax import lax
from jax.experimental import pallas as pl
from jax.experimental.pallas import tpu as pltpu
from jax.experimental.pallas import tpu_sc as plsc

VOCAB = 100000
DIM = 128
B = 4096
L = 50

NC = 2
NS = 16
NW = NC * NS

N_IDX = B * L
IDX_PER_W = N_IDX // NW
CHUNK = 128
CHUNKS_PER_W = IDX_PER_W // CHUNK


NBUF = 5
ROUNDS = CHUNKS_PER_W // NBUF


def _body(words_hbm, pre_hbm, trn_hbm, out_hbm, idx_v, *rest):
    bufs = rest[0:NBUF]
    semg = rest[NBUF : 2 * NBUF]
    sema = rest[2 * NBUF : 3 * NBUF]
    semo = rest[3 * NBUF : 4 * NBUF]

    wid = lax.axis_index("s") * NC + lax.axis_index("c")
    base = wid * IDX_PER_W

    pltpu.sync_copy(words_hbm.at[wid], idx_v)

    def fire_g1(j, b):
        pltpu.async_copy(pre_hbm.at[idx_v.at[j]], bufs[b], semg[b])

    def wait_g1(j, b):
        pltpu.make_async_copy(pre_hbm.at[idx_v.at[j]], bufs[b], semg[b]).wait()

    def process(j, b):
        cp = pltpu.async_copy(trn_hbm.at[idx_v.at[j]], bufs[b], sema[b], add=True)
        cp.wait()
        pltpu.async_copy(bufs[b], out_hbm.at[pl.ds(base + j * CHUNK, CHUNK)], semo[b])

    def wait_out(b):
        pltpu.make_async_copy(bufs[b], out_hbm.at[pl.ds(base, CHUNK)], semo[b]).wait()

    for b in range(NBUF):
        fire_g1(b, b)

    def round_body(g, carry):
        j0 = g * NBUF
        for b in range(NBUF):
            wait_g1(j0 + b, b)
            process(j0 + b, b)
        for b in range(NBUF):
            wait_out(b)
            fire_g1(j0 + b + NBUF, b)
        return carry

    lax.fori_loop(0, ROUNDS - 1, round_body, 0)

    j0 = (ROUNDS - 1) * NBUF
    for b in range(NBUF):
        wait_g1(j0 + b, b)
        process(j0 + b, b)
    for b in range(NBUF):
        wait_out(b)


@jax.jit
def _run(words2d, pre, trn):
    mesh = plsc.VectorSubcoreMesh(
        core_axis_name="c", subcore_axis_name="s", num_cores=NC, num_subcores=NS
    )
    f = pl.kernel(
        _body,
        out_type=jax.ShapeDtypeStruct((N_IDX, DIM), jnp.float32),
        mesh=mesh,
        scratch_types=(
            [pltpu.VMEM((CHUNKS_PER_W, CHUNK), jnp.int32)]
            + [pltpu.VMEM((CHUNK, DIM), jnp.float32) for _ in range(NBUF)]
            + [pltpu.SemaphoreType.DMA for _ in range(3 * NBUF)]
        ),
    )
    return f(words2d, pre, trn)


def kernel(words, pretrained_weight, trainable_weight):
    words2d = words.reshape(NW, CHUNKS_PER_W, CHUNK)
    out = _run(words2d, pretrained_weight, trainable_weight)
    return out.reshape(B, L, DIM)


# speedup vs baseline: 8.4562x; 1.6458x over previous
import jax
import jax.numpy as jnp
from jax import lax
from jax.experimental import pallas as pl
from jax.experimental.pallas import tpu as pltpu
from jax.experimental.pallas import tpu_sc as plsc

VOCAB = 100000
DIM = 128
B = 4096
L = 50

NC = 2
NS = 16
NW = NC * NS

BPW = B // NW
SUB = 4
CHUNKS = BPW // SUB
NBUF = 4
ROUNDS = CHUNKS // NBUF


def _body(words_hbm, pre_hbm, trn_hbm, out_hbm, idx_v, *rest):
    bufs = rest[0:NBUF]
    semg = rest[NBUF : 2 * NBUF]
    sema = rest[2 * NBUF : 3 * NBUF]
    semo = rest[3 * NBUF : 4 * NBUF]

    wid = lax.axis_index("s") * NC + lax.axis_index("c")
    brow0 = wid * BPW

    pltpu.sync_copy(words_hbm.at[pl.ds(brow0, BPW)], idx_v)

    def fire_g1(k, b):
        for m in range(SUB):
            pltpu.async_copy(pre_hbm.at[idx_v.at[k * SUB + m]], bufs[b].at[m], semg[b])

    def wait_g1(k, b):
        for m in range(SUB):
            pltpu.make_async_copy(
                pre_hbm.at[idx_v.at[k * SUB + m]], bufs[b].at[m], semg[b]
            ).wait()

    def process(k, b):
        for m in range(SUB):
            pltpu.async_copy(
                trn_hbm.at[idx_v.at[k * SUB + m]], bufs[b].at[m], sema[b], add=True
            )
        for m in range(SUB):
            pltpu.make_async_copy(
                trn_hbm.at[idx_v.at[k * SUB + m]], bufs[b].at[m], sema[b]
            ).wait()
        pltpu.async_copy(bufs[b], out_hbm.at[pl.ds(brow0 + k * SUB, SUB)], semo[b])

    def wait_out(b):
        pltpu.make_async_copy(bufs[b], out_hbm.at[pl.ds(brow0, SUB)], semo[b]).wait()

    for b in range(NBUF):
        fire_g1(b, b)

    def round_body(g, carry):
        k0 = g * NBUF
        for b in range(NBUF):
            wait_g1(k0 + b, b)
            process(k0 + b, b)
        for b in range(NBUF):
            wait_out(b)
            fire_g1(k0 + b + NBUF, b)
        return carry

    lax.fori_loop(0, ROUNDS - 1, round_body, 0)

    k0 = (ROUNDS - 1) * NBUF
    for b in range(NBUF):
        wait_g1(k0 + b, b)
        process(k0 + b, b)
    for b in range(NBUF):
        wait_out(b)


@jax.jit
def _run(words, pre, trn):
    mesh = plsc.VectorSubcoreMesh(
        core_axis_name="c", subcore_axis_name="s", num_cores=NC, num_subcores=NS
    )
    f = pl.kernel(
        _body,
        out_type=jax.ShapeDtypeStruct((B, L, DIM), jnp.float32),
        mesh=mesh,
        scratch_types=(
            [pltpu.VMEM((BPW, L), jnp.int32)]
            + [pltpu.VMEM((SUB, L, DIM), jnp.float32) for _ in range(NBUF)]
            + [pltpu.SemaphoreType.DMA for _ in range(3 * NBUF)]
        ),
    )
    return f(words, pre, trn)


def kernel(words, pretrained_weight, trainable_weight):
    return _run(words, pretrained_weight, trainable_weight)


# speedup vs baseline: 11.8656x; 1.4032x over previous
import jax
import jax.numpy as jnp
from jax import lax
from jax.experimental import pallas as pl
from jax.experimental.pallas import tpu as pltpu
from jax.experimental.pallas import tpu_sc as plsc

VOCAB = 100000
DIM = 128
B = 4096
L = 50

NC = 2
NS = 16
NW = NC * NS

BPW = B // NW
CHUNK = BPW
NBUF = 5
ROUNDS = L // NBUF


def _body(words_hbm, pre_hbm, trn_hbm, out_hbm, idx_v, *rest):
    bufs = rest[0:NBUF]
    semg = rest[NBUF : 2 * NBUF]
    sema = rest[2 * NBUF : 3 * NBUF]
    semo = rest[3 * NBUF : 4 * NBUF]

    wid = lax.axis_index("s") * NC + lax.axis_index("c")
    brow0 = wid * BPW

    pltpu.sync_copy(words_hbm.at[:, pl.ds(brow0, BPW)], idx_v)

    def fire_g1(l, b):
        pltpu.async_copy(pre_hbm.at[idx_v.at[l]], bufs[b], semg[b])

    def wait_g1(l, b):
        pltpu.make_async_copy(pre_hbm.at[idx_v.at[l]], bufs[b], semg[b]).wait()

    def process(l, b):
        cp = pltpu.async_copy(trn_hbm.at[idx_v.at[l]], bufs[b], sema[b], add=True)
        cp.wait()
        pltpu.async_copy(bufs[b], out_hbm.at[l, pl.ds(brow0, BPW)], semo[b])

    def wait_out(b):
        pltpu.make_async_copy(bufs[b], out_hbm.at[0, pl.ds(brow0, BPW)], semo[b]).wait()

    for b in range(NBUF):
        fire_g1(b, b)

    def round_body(g, carry):
        l0 = g * NBUF
        for b in range(NBUF):
            wait_g1(l0 + b, b)
            process(l0 + b, b)
        for b in range(NBUF):
            wait_out(b)
            fire_g1(l0 + b + NBUF, b)
        return carry

    lax.fori_loop(0, ROUNDS - 1, round_body, 0)

    l0 = (ROUNDS - 1) * NBUF
    for b in range(NBUF):
        wait_g1(l0 + b, b)
        process(l0 + b, b)
    for b in range(NBUF):
        wait_out(b)


@jax.jit
def _run(words, pre, trn):
    mesh = plsc.VectorSubcoreMesh(
        core_axis_name="c", subcore_axis_name="s", num_cores=NC, num_subcores=NS
    )
    f = pl.kernel(
        _body,
        out_type=jax.ShapeDtypeStruct((L, B, DIM), jnp.float32),
        mesh=mesh,
        scratch_types=(
            [pltpu.VMEM((L, BPW), jnp.int32)]
            + [pltpu.VMEM((CHUNK, DIM), jnp.float32) for _ in range(NBUF)]
            + [pltpu.SemaphoreType.DMA for _ in range(3 * NBUF)]
        ),
    )
    out_lbd = f(jnp.transpose(words), pre, trn)
    return jnp.transpose(out_lbd, (1, 0, 2))


def kernel(words, pretrained_weight, trainable_weight):
    return _run(words, pretrained_weight, trainable_weight)
